# Initial kernel scaffold; baseline (speedup 1.0000x reference)
import jax
import jax.numpy as jnp
from jax.experimental import pallas as pl


def kernel(sentiment, emb_table, W, b):
    raise NotImplementedError("write your pallas kernel here")


# speedup vs baseline: 4.0190x; 4.0190x over previous
import jax
import jax.numpy as jnp
from jax import lax
from jax.experimental import pallas as pl
from jax.experimental.pallas import tpu as pltpu
from jax.experimental.pallas import tpu_sc as plsc

_NUM_CLASSES = 1000
_EMB = 64
_NC = 2
_NS = 16
_NW = _NC * _NS
_K = 128
_SUB = 8
_CB = _K * _SUB


def _z_body(t_ref, w_ref, b_ref, z_ref):
    t = t_ref[...]
    row = lax.broadcasted_iota(jnp.int32, t.shape, 0)
    t = jnp.where(row == 0, jnp.float32(0.0), t)
    y = lax.dot_general(t, w_ref[...], (((1,), (1,)), ((), ())),
                        preferred_element_type=jnp.float32)
    z_ref[...] = jnp.tanh(y + b_ref[...])


def _compute_z(table, w, b):
    return pl.pallas_call(
        _z_body,
        out_shape=jax.ShapeDtypeStruct((_NUM_CLASSES, _EMB), jnp.float32),
    )(table, w, b.reshape(1, _EMB))


def _gather_body(z_hbm, idx_hbm, out_hbm, idx_v, rows_v, sem):
    wid = lax.axis_index("s") * _NC + lax.axis_index("c")
    n_chunks = idx_hbm.shape[0] // (_NW * _SUB)
    base_row = wid * (n_chunks * _SUB)

    def chunk(i, carry):
        row0 = base_row + i * _SUB
        pltpu.sync_copy(idx_hbm.at[pl.ds(row0, _SUB)], idx_v)
        copies = [
            pltpu.async_copy(z_hbm.at[idx_v.at[j]],
                             rows_v.at[pl.ds(j * _K, _K)], sem)
            for j in range(_SUB)
        ]
        for c in copies:
            c.wait()
        pltpu.sync_copy(rows_v, out_hbm.at[pl.ds(row0 * _K, _CB)])
        return carry

    lax.fori_loop(0, n_chunks, chunk, 0)


def _sc_gather(z, idx2):
    total = idx2.shape[0] * _K
    kfn = pl.kernel(
        _gather_body,
        out_type=jax.ShapeDtypeStruct((total, _EMB), jnp.float32),
        mesh=plsc.VectorSubcoreMesh(core_axis_name="c", subcore_axis_name="s"),
        scratch_types=[
            pltpu.VMEM((_SUB, _K), jnp.int32),
            pltpu.VMEM((_CB, _EMB), jnp.float32),
            pltpu.SemaphoreType.DMA,
        ],
        compiler_params=pltpu.CompilerParams(use_tc_tiling_on_sc=False),
    )
    return kfn(z, idx2)


def kernel(sentiment, emb_table, W, b):
    batch, hist = sentiment.shape
    z = _compute_z(emb_table, W, b)
    idx2 = sentiment.reshape((batch * hist) // _K, _K)
    out = _sc_gather(z, idx2)
    return out.reshape(batch, hist, _EMB)
